# baseline (speedup 1.0000x reference)
import functools

import jax
import jax.numpy as jnp
from jax.experimental import pallas as pl
from jax.experimental.pallas import tpu as pltpu


def _moe_dense_body(x_ref, wg_ref, w1_ref, w2_ref, o_ref, comb_ref, *, n_exp):
    e = pl.program_id(1)

    @pl.when(e == 0)
    def _():
        xf = x_ref[...]
        logits = jax.lax.dot_general(
            xf, wg_ref[...], (((1,), (1,)), ((), ())),
            preferred_element_type=jnp.float32)
        eidx = jax.lax.broadcasted_iota(jnp.int32, logits.shape, 1)
        m1 = jnp.max(logits, axis=1, keepdims=True)
        i1 = jnp.min(jnp.where(logits == m1, eidx, n_exp), axis=1, keepdims=True)
        masked = jnp.where(eidx == i1, -jnp.inf, logits)
        m2 = jnp.max(masked, axis=1, keepdims=True)
        i2 = jnp.min(jnp.where(masked == m2, eidx, n_exp), axis=1, keepdims=True)
        g1 = 1.0 / (1.0 + jnp.exp(m2 - m1))
        g2 = 1.0 - g1
        comb_ref[...] = (jnp.where(eidx == i1, g1, 0.0)
                         + jnp.where(eidx == i2, g2, 0.0))

    xb = x_ref[...].astype(jnp.bfloat16)
    h = jax.lax.dot_general(
        xb, w1_ref[0], (((1,), (1,)), ((), ())),
        preferred_element_type=jnp.float32)
    h = jnp.maximum(h, 0.0).astype(jnp.bfloat16)
    y = jax.lax.dot_general(
        h, w2_ref[0], (((1,), (1,)), ((), ())),
        preferred_element_type=jnp.float32)

    comb = comb_ref[...]
    eidx2 = jax.lax.broadcasted_iota(jnp.int32, comb.shape, 1)
    c = jnp.sum(jnp.where(eidx2 == e, comb, 0.0), axis=1, keepdims=True)
    contrib = c * y

    @pl.when(e == 0)
    def _():
        o_ref[...] = contrib

    @pl.when(e > 0)
    def _():
        o_ref[...] = o_ref[...] + contrib


def _moe_dense(x2, wg, w1b, w2b, *, t_blk):
    s, d = x2.shape
    n_exp, dff, _ = w1b.shape
    nt = s // t_blk
    grid = (nt, n_exp)
    return pl.pallas_call(
        functools.partial(_moe_dense_body, n_exp=n_exp),
        grid=grid,
        in_specs=[
            pl.BlockSpec((t_blk, d), lambda t, e: (t, 0)),
            pl.BlockSpec((n_exp, d), lambda t, e: (0, 0)),
            pl.BlockSpec((1, dff, d), lambda t, e: (e, 0, 0)),
            pl.BlockSpec((1, d, dff), lambda t, e: (e, 0, 0)),
        ],
        out_specs=pl.BlockSpec((t_blk, d), lambda t, e: (t, 0)),
        out_shape=jax.ShapeDtypeStruct((s, d), jnp.float32),
        scratch_shapes=[pltpu.VMEM((t_blk, n_exp), jnp.float32)],
        compiler_params=pltpu.CompilerParams(
            dimension_semantics=("parallel", "arbitrary")),
    )(x2, wg, w1b, w2b)


def kernel(x, Wg, W1, W2):
    b, s, d = x.shape
    x2 = x.reshape(s, d)
    w1b = W1.astype(jnp.bfloat16)
    w2b = W2.astype(jnp.bfloat16)
    out = _moe_dense(x2, Wg, w1b, w2b, t_blk=512)
    return out.reshape(b, s, d)


# speedup vs baseline: 1.6432x; 1.6432x over previous
import functools

import jax
import jax.numpy as jnp
from jax import lax
from jax.experimental import pallas as pl
from jax.experimental.pallas import tpu as pltpu
from jax.experimental.pallas import tpu_sc as plsc

S = 2048
D = 768
E = 8
DFF = 3072
BLK = 256
NBLK = (2 * S) // BLK + E
NPAD = NBLK * BLK
CHUNK = 128


def _route_body(x_ref, wg_ref, xg0_ref, xg1_ref, pos0_ref, pos1_ref,
                counts_ref, cex_ref):
    xf = x_ref[...]
    logits = lax.dot_general(xf, wg_ref[...], (((1,), (1,)), ((), ())),
                             preferred_element_type=jnp.float32)
    eidx = lax.broadcasted_iota(jnp.int32, (S, E), 1)
    m1 = jnp.max(logits, axis=1, keepdims=True)
    i1 = jnp.min(jnp.where(logits == m1, eidx, E), axis=1, keepdims=True)
    masked = jnp.where(eidx == i1, -jnp.inf, logits)
    m2 = jnp.max(masked, axis=1, keepdims=True)
    i2 = jnp.min(jnp.where(masked == m2, eidx, E), axis=1, keepdims=True)
    g1 = 1.0 / (1.0 + jnp.exp(m2 - m1))
    g2 = 1.0 - g1
    a1 = eidx == i1
    a2 = eidx == i2
    m = jnp.where(a1, 1.0, 0.0) + jnp.where(a2, 1.0, 0.0)

    r = lax.broadcasted_iota(jnp.int32, (CHUNK, CHUNK), 0)
    c = lax.broadcasted_iota(jnp.int32, (CHUNK, CHUNK), 1)
    ltri = jnp.where(r > c, 1.0, 0.0)
    carry = jnp.zeros((1, E), jnp.float32)
    for k in range(S // CHUNK):
        mc = m[k * CHUNK:(k + 1) * CHUNK, :]
        cex_ref[k * CHUNK:(k + 1) * CHUNK, :] = carry + lax.dot_general(
            ltri, mc, (((1,), (0,)), ((), ())),
            preferred_element_type=jnp.float32)
        carry = carry + jnp.sum(mc, axis=0, keepdims=True)

    counts = carry
    padded = jnp.ceil(counts / BLK) * BLK
    uidx_r = lax.broadcasted_iota(jnp.int32, (E, E), 0)
    uidx_c = lax.broadcasted_iota(jnp.int32, (E, E), 1)
    utri = jnp.where(uidx_r < uidx_c, 1.0, 0.0)
    start = lax.dot_general(padded, utri, (((1,), (0,)), ((), ())),
                            preferred_element_type=jnp.float32)
    base = start + cex_ref[...]
    pos0 = jnp.sum(jnp.where(a1, base, 0.0), axis=1, keepdims=True)
    pos1 = jnp.sum(jnp.where(a2, base, 0.0), axis=1, keepdims=True)
    pos0_ref[...] = pos0.astype(jnp.int32)
    pos1_ref[...] = pos1.astype(jnp.int32)
    counts_ref[...] = counts.astype(jnp.int32)
    xg0_ref[...] = g1 * xf
    xg1_ref[...] = g2 * xf


def _route(x2, wg):
    return pl.pallas_call(
        _route_body,
        out_shape=(
            jax.ShapeDtypeStruct((S, D), jnp.float32),
            jax.ShapeDtypeStruct((S, D), jnp.float32),
            jax.ShapeDtypeStruct((S, 1), jnp.int32),
            jax.ShapeDtypeStruct((S, 1), jnp.int32),
            jax.ShapeDtypeStruct((1, E), jnp.int32),
        ),
        scratch_shapes=[pltpu.VMEM((S, E), jnp.float32)],
    )(x2, wg)


def _make_scatter():
    info = plsc.get_sparse_core_info()
    nc, ns = info.num_cores, info.num_subcores
    nw = nc * ns
    rw = S // nw
    mesh = plsc.VectorSubcoreMesh(core_axis_name="c", subcore_axis_name="s")

    @functools.partial(
        pl.kernel, mesh=mesh,
        out_type=jax.ShapeDtypeStruct((NPAD, D), jnp.float32),
        scratch_types=[
            pltpu.VMEM((rw,), jnp.int32),
            pltpu.VMEM((rw,), jnp.int32),
            pltpu.VMEM((rw, D), jnp.float32),
            pltpu.VMEM((rw, D), jnp.float32),
            pltpu.SemaphoreType.DMA,
            pltpu.SemaphoreType.DMA,
        ],
    )
    def scatter_k(xg0_hbm, xg1_hbm, pos0_hbm, pos1_hbm, xs_hbm,
                  idx0_v, idx1_v, rows0_v, rows1_v, sem0, sem1):
        wid = lax.axis_index("s") * nc + lax.axis_index("c")
        b = wid * rw
        pltpu.sync_copy(pos0_hbm.at[pl.ds(b, rw)], idx0_v)
        pltpu.sync_copy(pos1_hbm.at[pl.ds(b, rw)], idx1_v)
        pltpu.sync_copy(xg0_hbm.at[pl.ds(b, rw)], rows0_v)
        pltpu.sync_copy(xg1_hbm.at[pl.ds(b, rw)], rows1_v)
        c0 = pltpu.async_copy(rows0_v, xs_hbm.at[idx0_v], sem0)
        c1 = pltpu.async_copy(rows1_v, xs_hbm.at[idx1_v], sem1)
        c0.wait()
        c1.wait()

    return scatter_k


def _ffn_body(bids_ref, used_ref, xs_ref, w1_ref, w2_ref, ys_ref):
    b = pl.program_id(0)

    @pl.when(used_ref[b] == 1)
    def _():
        xb = xs_ref[...]
        h = lax.dot_general(xb, w1_ref[0], (((1,), (1,)), ((), ())),
                            preferred_element_type=jnp.float32)
        h = jnp.maximum(h, 0.0)
        ys_ref[...] = lax.dot_general(h, w2_ref[0], (((1,), (1,)), ((), ())),
                                      preferred_element_type=jnp.float32)


def _ffn(bids, used, xs, w1, w2):
    grid_spec = pltpu.PrefetchScalarGridSpec(
        num_scalar_prefetch=2,
        grid=(NBLK,),
        in_specs=[
            pl.BlockSpec((BLK, D), lambda b, bids, used: (b, 0)),
            pl.BlockSpec((1, DFF, D), lambda b, bids, used: (bids[b], 0, 0)),
            pl.BlockSpec((1, D, DFF), lambda b, bids, used: (bids[b], 0, 0)),
        ],
        out_specs=pl.BlockSpec((BLK, D), lambda b, bids, used: (b, 0)),
    )
    return pl.pallas_call(
        _ffn_body,
        grid_spec=grid_spec,
        out_shape=jax.ShapeDtypeStruct((NPAD, D), jnp.float32),
        compiler_params=pltpu.CompilerParams(
            dimension_semantics=("arbitrary",)),
    )(bids, used, xs, w1, w2)


def _make_combine():
    info = plsc.get_sparse_core_info()
    nc, ns = info.num_cores, info.num_subcores
    nw = nc * ns
    rw = S // nw
    nv = D // 16
    mesh = plsc.VectorSubcoreMesh(core_axis_name="c", subcore_axis_name="s")

    @functools.partial(
        pl.kernel, mesh=mesh,
        out_type=jax.ShapeDtypeStruct((S, D), jnp.float32),
        scratch_types=[
            pltpu.VMEM((rw,), jnp.int32),
            pltpu.VMEM((rw,), jnp.int32),
            pltpu.VMEM((rw, D), jnp.float32),
            pltpu.VMEM((rw, D), jnp.float32),
            pltpu.SemaphoreType.DMA,
            pltpu.SemaphoreType.DMA,
        ],
    )
    def combine_k(ys_hbm, pos0_hbm, pos1_hbm, out_hbm,
                  idx0_v, idx1_v, bufa_v, bufb_v, sem0, sem1):
        wid = lax.axis_index("s") * nc + lax.axis_index("c")
        b = wid * rw
        pltpu.sync_copy(pos0_hbm.at[pl.ds(b, rw)], idx0_v)
        pltpu.sync_copy(pos1_hbm.at[pl.ds(b, rw)], idx1_v)
        ca = pltpu.async_copy(ys_hbm.at[idx0_v], bufa_v, sem0)
        cb = pltpu.async_copy(ys_hbm.at[idx1_v], bufb_v, sem1)
        ca.wait()
        cb.wait()

        def row(r, _):
            for cc in range(nv):
                sl = pl.ds(cc * 16, 16)
                bufa_v[r, sl] = bufa_v[r, sl] + bufb_v[r, sl]
            return 0

        lax.fori_loop(0, rw, row, 0)
        pltpu.sync_copy(bufa_v, out_hbm.at[pl.ds(b, rw)])

    return combine_k


def kernel(x, Wg, W1, W2):
    bsz, s, d = x.shape
    x2 = x.reshape(s, d)

    xg0, xg1, pos0, pos1, counts = _route(x2, Wg)
    pos0 = pos0.reshape(S)
    pos1 = pos1.reshape(S)
    counts = counts.reshape(E)

    nb = (counts + BLK - 1) // BLK
    cumnb = jnp.cumsum(nb)
    total = cumnb[-1]
    j = jnp.arange(NBLK, dtype=jnp.int32)
    bid_raw = jnp.sum((j[:, None] >= cumnb[None, :]).astype(jnp.int32), axis=1)
    used = (j < total).astype(jnp.int32)
    last_bid = jnp.max(jnp.where(nb > 0, jnp.arange(E, dtype=jnp.int32), 0))
    bids = jnp.where(used == 1, jnp.minimum(bid_raw, E - 1), last_bid)

    xs = _make_scatter()(xg0, xg1, pos0, pos1)
    ys = _ffn(bids, used, xs, W1, W2)
    out = _make_combine()(ys, pos0, pos1)
    return out.reshape(bsz, s, d)
